# Initial kernel scaffold; baseline (speedup 1.0000x reference)
import jax
import jax.numpy as jnp
from jax.experimental import pallas as pl


def kernel(x, edge_index, edge_weight, W1, b1, W2, b2):
    raise NotImplementedError("write your pallas kernel here")


# speedup vs baseline: 17.7310x; 17.7310x over previous
import functools

import jax
import jax.numpy as jnp
from jax import lax
from jax.experimental import pallas as pl
from jax.experimental.pallas import tpu as pltpu
from jax.experimental.pallas import tpu_sc as plsc

N = 10000
E = 320000
D_IN = 128
H = 128
Z = 64

NC = 2
NS = 16
NW = NC * NS
LANES = 16

CHUNK = 128
EPW = -(-E // (NW * CHUNK)) * CHUNK
NCH = EPW // CHUNK
EPAD = EPW * NW

ZPAD = 128
RBLK = 512
NPAD = -(-N // RBLK) * RBLK
ROWS_PER_TILE = NPAD // NS


def _worker_ids():
    cid = lax.axis_index("c")
    tid = lax.axis_index("s")
    wid = tid * NC + cid
    return cid, tid, wid


@functools.cache
def _sc_mesh():
    return plsc.VectorSubcoreMesh(core_axis_name="c", subcore_axis_name="s",
                                  num_cores=NC, num_subcores=NS)


@functools.cache
def _make_deg_kernel():
    @functools.partial(
        pl.kernel,
        out_type=jax.ShapeDtypeStruct((NC, NPAD), jnp.float32),
        scratch_types=[
            pltpu.VMEM((NCH, CHUNK), jnp.int32),
            pltpu.VMEM((NCH, CHUNK), jnp.float32),
            pltpu.VMEM((ROWS_PER_TILE,), jnp.float32),
            pltpu.VMEM_SHARED((NPAD,), jnp.float32),
        ],
        mesh=_sc_mesh(),
    )
    def _deg_kernel(dst_hbm, ew_hbm, out_hbm, idx_v, ew_v, zero_v, acc_shared):
        cid, tid, wid = _worker_ids()

        def zbody(i, _):
            zero_v[pl.ds(i * LANES, LANES)] = jnp.zeros((LANES,), jnp.float32)
            return 0

        lax.fori_loop(0, ROWS_PER_TILE // LANES, zbody, 0)
        pltpu.sync_copy(zero_v,
                        acc_shared.at[pl.ds(tid * ROWS_PER_TILE, ROWS_PER_TILE)])
        plsc.subcore_barrier()

        pltpu.sync_copy(dst_hbm.at[wid], idx_v)
        pltpu.sync_copy(ew_hbm.at[wid], ew_v)

        def body(j, _):
            pltpu.sync_copy(ew_v.at[j], acc_shared.at[idx_v.at[j]], add=True)
            return 0

        lax.fori_loop(0, NCH, body, 0)
        plsc.subcore_barrier()

        pltpu.sync_copy(
            acc_shared.at[pl.ds(tid * ROWS_PER_TILE, ROWS_PER_TILE)],
            out_hbm.at[cid, pl.ds(tid * ROWS_PER_TILE, ROWS_PER_TILE)],
        )

    return _deg_kernel


@functools.cache
def _make_agg_kernel(D):
    @functools.partial(
        pl.kernel,
        out_type=jax.ShapeDtypeStruct((NC, NPAD, D), jnp.float32),
        scratch_types=[
            pltpu.VMEM((NCH, CHUNK), jnp.int32),
            pltpu.VMEM((NCH, CHUNK), jnp.int32),
            pltpu.VMEM((NCH, CHUNK), jnp.float32),
            pltpu.VMEM((CHUNK, D), jnp.float32),
            pltpu.VMEM_SHARED((NPAD, D), jnp.float32),
            pltpu.SemaphoreType.DMA,
        ],
        mesh=_sc_mesh(),
    )
    def agg(s_hbm, src_hbm, dst_hbm, ew_hbm, out_hbm,
            src_v, dst_v, ew_v, rows_v, acc_shared, sem):
        cid, tid, wid = _worker_ids()

        def zbody(r, _):
            for d in range(D // LANES):
                rows_v[r, pl.ds(d * LANES, LANES)] = jnp.zeros((LANES,), jnp.float32)
            return 0

        lax.fori_loop(0, CHUNK, zbody, 0)

        def zcopy(k, _):
            pltpu.sync_copy(
                rows_v,
                acc_shared.at[pl.ds(tid * ROWS_PER_TILE + k * CHUNK, CHUNK)],
            )
            return 0

        lax.fori_loop(0, ROWS_PER_TILE // CHUNK, zcopy, 0)
        plsc.subcore_barrier()

        pltpu.sync_copy(src_hbm.at[wid], src_v)
        pltpu.sync_copy(dst_hbm.at[wid], dst_v)
        pltpu.sync_copy(ew_hbm.at[wid], ew_v)

        lane_ids = [jnp.full((LANES, 1), r, jnp.int32) for r in range(LANES)]
        dnums = lax.GatherDimensionNumbers(
            offset_dims=(), collapsed_slice_dims=(0,), start_index_map=(0,))

        def bcast_lane(wvec, r):
            return lax.gather(wvec, lane_ids[r], dnums, (1,),
                              mode=lax.GatherScatterMode.PROMISE_IN_BOUNDS)

        def body(j, _):
            pltpu.async_copy(s_hbm.at[src_v.at[j]], rows_v, sem).wait()

            def scale(g, _):
                wvec = ew_v[j, pl.ds(g * LANES, LANES)]
                for r in range(LANES):
                    w = bcast_lane(wvec, r)
                    row = g * LANES + r
                    for d in range(D // LANES):
                        sl = pl.ds(d * LANES, LANES)
                        rows_v[row, sl] = rows_v[row, sl] * w
                return 0

            lax.fori_loop(0, CHUNK // LANES, scale, 0)
            pltpu.sync_copy(rows_v, acc_shared.at[dst_v.at[j]], add=True)
            return 0

        lax.fori_loop(0, NCH, body, 0)
        plsc.subcore_barrier()

        pltpu.sync_copy(
            acc_shared.at[pl.ds(tid * ROWS_PER_TILE, ROWS_PER_TILE)],
            out_hbm.at[cid, pl.ds(tid * ROWS_PER_TILE, ROWS_PER_TILE)],
        )

    return agg


def _mm1_body(x_ref, w_ref, deg_ref, out_ref):
    dinv = lax.rsqrt(1.0 + deg_ref[0] + deg_ref[1])
    out_ref[...] = dinv * jnp.dot(x_ref[...], w_ref[...],
                                  preferred_element_type=jnp.float32)


def _mm2_body(p_ref, s1_ref, deg_ref, w_ref, b_ref, out_ref):
    dinv = lax.rsqrt(1.0 + deg_ref[0] + deg_ref[1])
    g = jnp.maximum(dinv * (p_ref[0] + p_ref[1] + s1_ref[...]) + b_ref[...], 0.0)
    out_ref[...] = dinv * jnp.dot(g, w_ref[...], preferred_element_type=jnp.float32)


def _fin_body(q_ref, s2_ref, deg_ref, b_ref, out_ref):
    dinv = lax.rsqrt(1.0 + deg_ref[0] + deg_ref[1])
    out_ref[...] = jnp.tanh(dinv * (q_ref[0] + q_ref[1] + s2_ref[...]) + b_ref[...])


def _tc_calls(xp, W1, b1, W2, b2, degp):
    grid = (NPAD // RBLK,)
    deg3 = degp.reshape(NC, NPAD, 1)
    dspec = pl.BlockSpec((NC, RBLK, 1), lambda i: (0, i, 0))

    mm1 = pl.pallas_call(
        _mm1_body,
        grid=grid,
        in_specs=[
            pl.BlockSpec((RBLK, D_IN), lambda i: (i, 0)),
            pl.BlockSpec((D_IN, H), lambda i: (0, 0)),
            dspec,
        ],
        out_specs=pl.BlockSpec((RBLK, H), lambda i: (i, 0)),
        out_shape=jax.ShapeDtypeStruct((NPAD, H), jnp.float32),
    )

    mm2 = pl.pallas_call(
        _mm2_body,
        grid=grid,
        in_specs=[
            pl.BlockSpec((NC, RBLK, H), lambda i: (0, i, 0)),
            pl.BlockSpec((RBLK, H), lambda i: (i, 0)),
            dspec,
            pl.BlockSpec((H, ZPAD), lambda i: (0, 0)),
            pl.BlockSpec((1, H), lambda i: (0, 0)),
        ],
        out_specs=pl.BlockSpec((RBLK, ZPAD), lambda i: (i, 0)),
        out_shape=jax.ShapeDtypeStruct((NPAD, ZPAD), jnp.float32),
    )

    fin = pl.pallas_call(
        _fin_body,
        grid=grid,
        in_specs=[
            pl.BlockSpec((NC, RBLK, ZPAD), lambda i: (0, i, 0)),
            pl.BlockSpec((RBLK, ZPAD), lambda i: (i, 0)),
            dspec,
            pl.BlockSpec((1, ZPAD), lambda i: (0, 0)),
        ],
        out_specs=pl.BlockSpec((RBLK, ZPAD), lambda i: (i, 0)),
        out_shape=jax.ShapeDtypeStruct((NPAD, ZPAD), jnp.float32),
    )
    return mm1, mm2, fin, deg3


@jax.jit
def kernel(x, edge_index, edge_weight, W1, b1, W2, b2):
    src = edge_index[0]
    dst = edge_index[1]

    pad = EPAD - E
    pad_idx = (jnp.arange(pad, dtype=jnp.int32) * 61) % N
    src3 = jnp.concatenate([src, pad_idx]).reshape(NW, NCH, CHUNK)
    dst3 = jnp.concatenate([dst, pad_idx]).reshape(NW, NCH, CHUNK)
    ew3 = jnp.concatenate(
        [edge_weight, jnp.zeros((pad,), jnp.float32)]).reshape(NW, NCH, CHUNK)

    xp = jnp.pad(x, ((0, NPAD - N), (0, 0)))

    W2p = jnp.pad(W2, ((0, 0), (0, ZPAD - Z)))
    b2p = jnp.pad(b2, (0, ZPAD - Z)).reshape(1, ZPAD)

    degp = _make_deg_kernel()(dst3, ew3)
    mm1, mm2, fin, deg3 = _tc_calls(xp, W1, b1, W2, b2, degp)

    s1 = mm1(xp, W1, deg3)
    p = _make_agg_kernel(H)(s1, src3, dst3, ew3)
    s2 = mm2(p, s1, deg3, W2p, b1.reshape(1, H))
    q = _make_agg_kernel(ZPAD)(s2, src3, dst3, ew3)
    z = fin(q, s2, deg3, b2p)
    return z[:N, :Z]


# speedup vs baseline: 22.9688x; 1.2954x over previous
import functools

import jax
import jax.numpy as jnp
from jax import lax
from jax.experimental import pallas as pl
from jax.experimental.pallas import tpu as pltpu
from jax.experimental.pallas import tpu_sc as plsc

N = 10000
E = 320000
D_IN = 128
H = 128
Z = 64

NC = 2
NS = 16
NW = NC * NS
LANES = 16

CHUNK = 128
NCH = 80
EPW = NCH * CHUNK
EPAD = EPW * NW

ZPAD = 128
RBLK = 512
NPAD = -(-N // RBLK) * RBLK
ROWS_PER_TILE = NPAD // NS


def _worker_ids():
    cid = lax.axis_index("c")
    tid = lax.axis_index("s")
    wid = tid * NC + cid
    return cid, tid, wid


@functools.cache
def _sc_mesh():
    return plsc.VectorSubcoreMesh(core_axis_name="c", subcore_axis_name="s",
                                  num_cores=NC, num_subcores=NS)


@functools.cache
def _make_deg_kernel():
    @functools.partial(
        pl.kernel,
        out_type=jax.ShapeDtypeStruct((NC, NPAD), jnp.float32),
        scratch_types=[
            pltpu.VMEM((NCH, CHUNK), jnp.int32),
            pltpu.VMEM((NCH, CHUNK), jnp.float32),
            pltpu.VMEM((ROWS_PER_TILE,), jnp.float32),
            pltpu.VMEM_SHARED((NPAD,), jnp.float32),
        ],
        mesh=_sc_mesh(),
    )
    def _deg_kernel(dst_hbm, ew_hbm, out_hbm, idx_v, ew_v, zero_v, acc_shared):
        cid, tid, wid = _worker_ids()

        def zbody(i, _):
            zero_v[pl.ds(i * LANES, LANES)] = jnp.zeros((LANES,), jnp.float32)
            return 0

        lax.fori_loop(0, ROWS_PER_TILE // LANES, zbody, 0)
        pltpu.sync_copy(zero_v,
                        acc_shared.at[pl.ds(tid * ROWS_PER_TILE, ROWS_PER_TILE)])
        plsc.subcore_barrier()

        pltpu.sync_copy(dst_hbm.at[wid], idx_v)
        pltpu.sync_copy(ew_hbm.at[wid], ew_v)

        def body(j, _):
            pltpu.sync_copy(ew_v.at[j], acc_shared.at[idx_v.at[j]], add=True)
            return 0

        lax.fori_loop(0, NCH, body, 0)
        plsc.subcore_barrier()

        pltpu.sync_copy(
            acc_shared.at[pl.ds(tid * ROWS_PER_TILE, ROWS_PER_TILE)],
            out_hbm.at[cid, pl.ds(tid * ROWS_PER_TILE, ROWS_PER_TILE)],
        )

    return _deg_kernel


@functools.cache
def _make_agg_kernel(D):
    @functools.partial(
        pl.kernel,
        out_type=jax.ShapeDtypeStruct((NC, NPAD, D), jnp.float32),
        scratch_types=[
            pltpu.VMEM((2, CHUNK), jnp.int32),
            pltpu.VMEM((2, CHUNK), jnp.int32),
            pltpu.VMEM((2, CHUNK), jnp.float32),
            pltpu.VMEM((2, CHUNK, D), jnp.float32),
            pltpu.VMEM_SHARED((NPAD, D), jnp.float32),
            pltpu.SemaphoreType.DMA,
            pltpu.SemaphoreType.DMA,
            pltpu.SemaphoreType.DMA,
            pltpu.SemaphoreType.DMA,
            pltpu.SemaphoreType.DMA,
            pltpu.SemaphoreType.DMA,
        ],
        mesh=_sc_mesh(),
    )
    def agg(s_hbm, src_hbm, dst_hbm, ew_hbm, out_hbm,
            esrc, edst, eew, rows, acc_shared,
            e_sem0, e_sem1, g_sem0, g_sem1, s_sem0, s_sem1):
        cid, tid, wid = _worker_ids()
        e_sems = (e_sem0, e_sem1)
        g_sems = (g_sem0, g_sem1)
        s_sems = (s_sem0, s_sem1)

        def eload(c, s):
            pltpu.async_copy(src_hbm.at[wid, c], esrc.at[s], e_sems[s])
            pltpu.async_copy(dst_hbm.at[wid, c], edst.at[s], e_sems[s])
            pltpu.async_copy(ew_hbm.at[wid, c], eew.at[s], e_sems[s])

        def ewait(s):
            pltpu.make_async_copy(src_hbm.at[wid, 0], esrc.at[s], e_sems[s]).wait()
            pltpu.make_async_copy(dst_hbm.at[wid, 0], edst.at[s], e_sems[s]).wait()
            pltpu.make_async_copy(ew_hbm.at[wid, 0], eew.at[s], e_sems[s]).wait()

        def gstart(s):
            pltpu.async_copy(s_hbm.at[esrc.at[s]], rows.at[s], g_sems[s])

        def gwait(s):
            pltpu.make_async_copy(s_hbm.at[esrc.at[s]], rows.at[s],
                                  g_sems[s]).wait()

        def sstart(s):
            pltpu.async_copy(rows.at[s], acc_shared.at[edst.at[s]], s_sems[s],
                             add=True)

        def swait(s):
            pltpu.make_async_copy(rows.at[s], acc_shared.at[edst.at[s]],
                                  s_sems[s]).wait()

        eload(0, 0)
        eload(1, 1)

        def zbody(r, _):
            for d in range(D // LANES):
                rows[0, r, pl.ds(d * LANES, LANES)] = jnp.zeros((LANES,),
                                                                jnp.float32)
            return 0

        lax.fori_loop(0, CHUNK, zbody, 0)

        def zcopy(k, _):
            pltpu.sync_copy(
                rows.at[0],
                acc_shared.at[pl.ds(tid * ROWS_PER_TILE + k * CHUNK, CHUNK)],
            )
            return 0

        lax.fori_loop(0, ROWS_PER_TILE // CHUNK, zcopy, 0)
        plsc.subcore_barrier()

        ewait(0)
        gstart(0)
        ewait(1)
        gstart(1)

        lane_ids = [jnp.full((LANES, 1), r, jnp.int32) for r in range(LANES)]
        dnums = lax.GatherDimensionNumbers(
            offset_dims=(), collapsed_slice_dims=(0,), start_index_map=(0,))

        def bcast_lane(wvec, r):
            return lax.gather(wvec, lane_ids[r], dnums, (1,),
                              mode=lax.GatherScatterMode.PROMISE_IN_BOUNDS)

        def scale(s):
            def grp(g, _):
                wvec = eew[s, pl.ds(g * LANES, LANES)]
                for r in range(LANES):
                    w = bcast_lane(wvec, r)
                    row = g * LANES + r
                    for d in range(D // LANES):
                        sl = pl.ds(d * LANES, LANES)
                        rows[s, row, sl] = rows[s, row, sl] * w
                return 0

            lax.fori_loop(0, CHUNK // LANES, grp, 0)

        def body(k, _):
            for s in (0, 1):
                gwait(s)
                scale(s)
                sstart(s)

            @pl.when(k < NCH // 2 - 1)
            def _prefetch():
                for s in (0, 1):
                    swait(s)
                    eload(2 * k + 2 + s, s)
                    ewait(s)
                    gstart(s)

            return 0

        lax.fori_loop(0, NCH // 2, body, 0)
        swait(0)
        swait(1)
        plsc.subcore_barrier()

        pltpu.sync_copy(
            acc_shared.at[pl.ds(tid * ROWS_PER_TILE, ROWS_PER_TILE)],
            out_hbm.at[cid, pl.ds(tid * ROWS_PER_TILE, ROWS_PER_TILE)],
        )

    return agg


def _mm1_body(x_ref, w_ref, deg_ref, out_ref):
    dinv = lax.rsqrt(1.0 + deg_ref[0] + deg_ref[1])
    out_ref[...] = dinv * jnp.dot(x_ref[...], w_ref[...],
                                  preferred_element_type=jnp.float32)


def _mm2_body(p_ref, s1_ref, deg_ref, w_ref, b_ref, out_ref):
    dinv = lax.rsqrt(1.0 + deg_ref[0] + deg_ref[1])
    g = jnp.maximum(dinv * (p_ref[0] + p_ref[1] + s1_ref[...]) + b_ref[...], 0.0)
    out_ref[...] = dinv * jnp.dot(g, w_ref[...], preferred_element_type=jnp.float32)


def _fin_body(q_ref, s2_ref, deg_ref, b_ref, out_ref):
    dinv = lax.rsqrt(1.0 + deg_ref[0] + deg_ref[1])
    out_ref[...] = jnp.tanh(dinv * (q_ref[0] + q_ref[1] + s2_ref[...]) + b_ref[...])


def _tc_calls(xp, W1, b1, W2, b2, degp):
    grid = (NPAD // RBLK,)
    deg3 = degp.reshape(NC, NPAD, 1)
    dspec = pl.BlockSpec((NC, RBLK, 1), lambda i: (0, i, 0))

    mm1 = pl.pallas_call(
        _mm1_body,
        grid=grid,
        in_specs=[
            pl.BlockSpec((RBLK, D_IN), lambda i: (i, 0)),
            pl.BlockSpec((D_IN, H), lambda i: (0, 0)),
            dspec,
        ],
        out_specs=pl.BlockSpec((RBLK, H), lambda i: (i, 0)),
        out_shape=jax.ShapeDtypeStruct((NPAD, H), jnp.float32),
    )

    mm2 = pl.pallas_call(
        _mm2_body,
        grid=grid,
        in_specs=[
            pl.BlockSpec((NC, RBLK, H), lambda i: (0, i, 0)),
            pl.BlockSpec((RBLK, H), lambda i: (i, 0)),
            dspec,
            pl.BlockSpec((H, ZPAD), lambda i: (0, 0)),
            pl.BlockSpec((1, H), lambda i: (0, 0)),
        ],
        out_specs=pl.BlockSpec((RBLK, ZPAD), lambda i: (i, 0)),
        out_shape=jax.ShapeDtypeStruct((NPAD, ZPAD), jnp.float32),
    )

    fin = pl.pallas_call(
        _fin_body,
        grid=grid,
        in_specs=[
            pl.BlockSpec((NC, RBLK, ZPAD), lambda i: (0, i, 0)),
            pl.BlockSpec((RBLK, ZPAD), lambda i: (i, 0)),
            dspec,
            pl.BlockSpec((1, ZPAD), lambda i: (0, 0)),
        ],
        out_specs=pl.BlockSpec((RBLK, ZPAD), lambda i: (i, 0)),
        out_shape=jax.ShapeDtypeStruct((NPAD, ZPAD), jnp.float32),
    )
    return mm1, mm2, fin, deg3


@jax.jit
def kernel(x, edge_index, edge_weight, W1, b1, W2, b2):
    src = edge_index[0]
    dst = edge_index[1]

    pad = EPAD - E
    pad_idx = (jnp.arange(pad, dtype=jnp.int32) * 61) % N
    src3 = jnp.concatenate([src, pad_idx]).reshape(NW, NCH, CHUNK)
    dst3 = jnp.concatenate([dst, pad_idx]).reshape(NW, NCH, CHUNK)
    ew3 = jnp.concatenate(
        [edge_weight, jnp.zeros((pad,), jnp.float32)]).reshape(NW, NCH, CHUNK)

    xp = jnp.pad(x, ((0, NPAD - N), (0, 0)))

    W2p = jnp.pad(W2, ((0, 0), (0, ZPAD - Z)))
    b2p = jnp.pad(b2, (0, ZPAD - Z)).reshape(1, ZPAD)

    degp = _make_deg_kernel()(dst3, ew3)
    mm1, mm2, fin, deg3 = _tc_calls(xp, W1, b1, W2, b2, degp)

    s1 = mm1(xp, W1, deg3)
    p = _make_agg_kernel(H)(s1, src3, dst3, ew3)
    s2 = mm2(p, s1, deg3, W2p, b1.reshape(1, H))
    q = _make_agg_kernel(ZPAD)(s2, src3, dst3, ew3)
    z = fin(q, s2, deg3, b2p)
    return z[:N, :Z]


# speedup vs baseline: 27.0863x; 1.1793x over previous
import functools

import jax
import jax.numpy as jnp
from jax import lax
from jax.experimental import pallas as pl
from jax.experimental.pallas import tpu as pltpu
from jax.experimental.pallas import tpu_sc as plsc

N = 10000
E = 320000
D_IN = 128
H = 128
Z = 64

NC = 2
NS = 16
NW = NC * NS
LANES = 16

CHUNK = 80
NCH = 128
EPW = NCH * CHUNK
EPAD = EPW * NW
K_ROWS = 4
K_EDGE = 8

ZPAD = 128
RBLK = 512
NPAD = -(-N // RBLK) * RBLK
ROWS_PER_TILE = NPAD // NS


def _worker_ids():
    cid = lax.axis_index("c")
    tid = lax.axis_index("s")
    wid = tid * NC + cid
    return cid, tid, wid


@functools.cache
def _sc_mesh():
    return plsc.VectorSubcoreMesh(core_axis_name="c", subcore_axis_name="s",
                                  num_cores=NC, num_subcores=NS)


@functools.cache
def _make_deg_kernel():
    @functools.partial(
        pl.kernel,
        out_type=jax.ShapeDtypeStruct((NC, NPAD), jnp.float32),
        scratch_types=[
            pltpu.VMEM((NCH, CHUNK), jnp.int32),
            pltpu.VMEM((NCH, CHUNK), jnp.float32),
            pltpu.VMEM((ROWS_PER_TILE,), jnp.float32),
            pltpu.VMEM_SHARED((NPAD,), jnp.float32),
        ],
        mesh=_sc_mesh(),
    )
    def _deg_kernel(dst_hbm, ew_hbm, out_hbm, idx_v, ew_v, zero_v, acc_shared):
        cid, tid, wid = _worker_ids()

        def zbody(i, _):
            zero_v[pl.ds(i * LANES, LANES)] = jnp.zeros((LANES,), jnp.float32)
            return 0

        lax.fori_loop(0, ROWS_PER_TILE // LANES, zbody, 0)
        pltpu.sync_copy(zero_v,
                        acc_shared.at[pl.ds(tid * ROWS_PER_TILE, ROWS_PER_TILE)])
        plsc.subcore_barrier()

        pltpu.sync_copy(dst_hbm.at[wid], idx_v)
        pltpu.sync_copy(ew_hbm.at[wid], ew_v)

        def body(j, _):
            pltpu.sync_copy(ew_v.at[j], acc_shared.at[idx_v.at[j]], add=True)
            return 0

        lax.fori_loop(0, NCH, body, 0)
        plsc.subcore_barrier()

        pltpu.sync_copy(
            acc_shared.at[pl.ds(tid * ROWS_PER_TILE, ROWS_PER_TILE)],
            out_hbm.at[cid, pl.ds(tid * ROWS_PER_TILE, ROWS_PER_TILE)],
        )

    return _deg_kernel


@functools.cache
def _make_agg_kernel(D):
    @functools.partial(
        pl.kernel,
        out_type=jax.ShapeDtypeStruct((NC, NPAD, D), jnp.float32),
        scratch_types=[
            pltpu.VMEM((K_EDGE, CHUNK), jnp.int32),
            pltpu.VMEM((K_EDGE, CHUNK), jnp.int32),
            pltpu.VMEM((K_EDGE, CHUNK), jnp.float32),
            pltpu.VMEM((K_ROWS, CHUNK, D), jnp.float32),
            pltpu.VMEM_SHARED((NPAD, D), jnp.float32),
            pltpu.SemaphoreType.DMA((K_EDGE,)),
            pltpu.SemaphoreType.DMA((K_ROWS,)),
            pltpu.SemaphoreType.DMA((K_ROWS,)),
        ],
        mesh=_sc_mesh(),
    )
    def agg(s_hbm, src_hbm, dst_hbm, ew_hbm, out_hbm,
            esrc, edst, eew, rows, acc_shared, e_sem, g_sem, s_sem):
        cid, tid, wid = _worker_ids()

        def eload(c, q):
            pltpu.async_copy(src_hbm.at[wid, c], esrc.at[q], e_sem.at[q])
            pltpu.async_copy(dst_hbm.at[wid, c], edst.at[q], e_sem.at[q])
            pltpu.async_copy(ew_hbm.at[wid, c], eew.at[q], e_sem.at[q])

        def ewait(q):
            pltpu.make_async_copy(src_hbm.at[wid, 0], esrc.at[q],
                                  e_sem.at[q]).wait()
            pltpu.make_async_copy(dst_hbm.at[wid, 0], edst.at[q],
                                  e_sem.at[q]).wait()
            pltpu.make_async_copy(ew_hbm.at[wid, 0], eew.at[q],
                                  e_sem.at[q]).wait()

        def gstart(q, r):
            pltpu.async_copy(s_hbm.at[esrc.at[q]], rows.at[r], g_sem.at[r])

        def gwait(q, r):
            pltpu.make_async_copy(s_hbm.at[esrc.at[q]], rows.at[r],
                                  g_sem.at[r]).wait()

        def sstart(q, r):
            pltpu.async_copy(rows.at[r], acc_shared.at[edst.at[q]],
                             s_sem.at[r], add=True)

        def swait(q, r):
            pltpu.make_async_copy(rows.at[r], acc_shared.at[edst.at[q]],
                                  s_sem.at[r]).wait()

        for q in range(5):
            eload(q, q)

        def zbody(r, _):
            for d in range(D // LANES):
                rows[0, r, pl.ds(d * LANES, LANES)] = jnp.zeros((LANES,),
                                                                jnp.float32)
            return 0

        lax.fori_loop(0, CHUNK, zbody, 0)

        def zcopy(k, _):
            pltpu.sync_copy(
                rows.at[0],
                acc_shared.at[pl.ds(tid * ROWS_PER_TILE + k * CHUNK, CHUNK)],
            )
            return 0

        lax.fori_loop(0, ROWS_PER_TILE // CHUNK, zcopy, 0)
        plsc.subcore_barrier()

        ewait(0)
        gstart(0, 0)
        ewait(1)
        gstart(1, 1)

        lane_ids = [jnp.full((LANES, 1), r, jnp.int32) for r in range(LANES)]
        dnums = lax.GatherDimensionNumbers(
            offset_dims=(), collapsed_slice_dims=(0,), start_index_map=(0,))

        def bcast_lane(wvec, r):
            return lax.gather(wvec, lane_ids[r], dnums, (1,),
                              mode=lax.GatherScatterMode.PROMISE_IN_BOUNDS)

        def scale(q):
            def grp(g, _):
                wvec = eew[q, pl.ds(g * LANES, LANES)]
                for r in range(LANES):
                    w = bcast_lane(wvec, r)
                    row = g * LANES + r
                    for d in range(D // LANES):
                        sl = pl.ds(d * LANES, LANES)
                        rows[q % K_ROWS, row, sl] = rows[q % K_ROWS, row, sl] * w
                return 0

            lax.fori_loop(0, CHUNK // LANES, grp, 0)

        def body(k, _):
            for i in range(K_EDGE):
                c = K_EDGE * k + i
                r = i % K_ROWS
                gwait(i, r)
                scale(i)
                sstart(i, r)

                if i >= 2:
                    swait(i - 2, (i - 2) % K_ROWS)
                else:
                    @pl.when(k > 0)
                    def _sw():
                        swait((i - 2) % K_EDGE, (i - 2) % K_ROWS)

                @pl.when(c + 5 < NCH)
                def _el():
                    eload(c + 5, (i + 5) % K_EDGE)

                @pl.when(c + 2 < NCH)
                def _gs():
                    ewait((i + 2) % K_EDGE)
                    gstart((i + 2) % K_EDGE, (i + 2) % K_ROWS)

            return 0

        lax.fori_loop(0, NCH // K_EDGE, body, 0)
        swait(K_EDGE - 2, (K_EDGE - 2) % K_ROWS)
        swait(K_EDGE - 1, (K_EDGE - 1) % K_ROWS)
        plsc.subcore_barrier()

        pltpu.sync_copy(
            acc_shared.at[pl.ds(tid * ROWS_PER_TILE, ROWS_PER_TILE)],
            out_hbm.at[cid, pl.ds(tid * ROWS_PER_TILE, ROWS_PER_TILE)],
        )

    return agg


def _mm1_body(x_ref, w_ref, deg_ref, out_ref):
    dinv = lax.rsqrt(1.0 + deg_ref[0] + deg_ref[1])
    out_ref[...] = dinv * jnp.dot(x_ref[...], w_ref[...],
                                  preferred_element_type=jnp.float32)


def _mm2_body(p_ref, s1_ref, deg_ref, w_ref, b_ref, out_ref):
    dinv = lax.rsqrt(1.0 + deg_ref[0] + deg_ref[1])
    g = jnp.maximum(dinv * (p_ref[0] + p_ref[1] + s1_ref[...]) + b_ref[...], 0.0)
    out_ref[...] = dinv * jnp.dot(g, w_ref[...], preferred_element_type=jnp.float32)


def _fin_body(q_ref, s2_ref, deg_ref, b_ref, out_ref):
    dinv = lax.rsqrt(1.0 + deg_ref[0] + deg_ref[1])
    out_ref[...] = jnp.tanh(dinv * (q_ref[0] + q_ref[1] + s2_ref[...]) + b_ref[...])


def _tc_calls(xp, W1, b1, W2, b2, degp):
    grid = (NPAD // RBLK,)
    deg3 = degp.reshape(NC, NPAD, 1)
    dspec = pl.BlockSpec((NC, RBLK, 1), lambda i: (0, i, 0))

    mm1 = pl.pallas_call(
        _mm1_body,
        grid=grid,
        in_specs=[
            pl.BlockSpec((RBLK, D_IN), lambda i: (i, 0)),
            pl.BlockSpec((D_IN, H), lambda i: (0, 0)),
            dspec,
        ],
        out_specs=pl.BlockSpec((RBLK, H), lambda i: (i, 0)),
        out_shape=jax.ShapeDtypeStruct((NPAD, H), jnp.float32),
    )

    mm2 = pl.pallas_call(
        _mm2_body,
        grid=grid,
        in_specs=[
            pl.BlockSpec((NC, RBLK, H), lambda i: (0, i, 0)),
            pl.BlockSpec((RBLK, H), lambda i: (i, 0)),
            dspec,
            pl.BlockSpec((H, ZPAD), lambda i: (0, 0)),
            pl.BlockSpec((1, H), lambda i: (0, 0)),
        ],
        out_specs=pl.BlockSpec((RBLK, ZPAD), lambda i: (i, 0)),
        out_shape=jax.ShapeDtypeStruct((NPAD, ZPAD), jnp.float32),
    )

    fin = pl.pallas_call(
        _fin_body,
        grid=grid,
        in_specs=[
            pl.BlockSpec((NC, RBLK, ZPAD), lambda i: (0, i, 0)),
            pl.BlockSpec((RBLK, ZPAD), lambda i: (i, 0)),
            dspec,
            pl.BlockSpec((1, ZPAD), lambda i: (0, 0)),
        ],
        out_specs=pl.BlockSpec((RBLK, ZPAD), lambda i: (i, 0)),
        out_shape=jax.ShapeDtypeStruct((NPAD, ZPAD), jnp.float32),
    )
    return mm1, mm2, fin, deg3


@jax.jit
def kernel(x, edge_index, edge_weight, W1, b1, W2, b2):
    src = edge_index[0]
    dst = edge_index[1]

    pad = EPAD - E
    pad_idx = (jnp.arange(pad, dtype=jnp.int32) * 61) % N
    src3 = jnp.concatenate([src, pad_idx]).reshape(NW, NCH, CHUNK)
    dst3 = jnp.concatenate([dst, pad_idx]).reshape(NW, NCH, CHUNK)
    ew3 = jnp.concatenate(
        [edge_weight, jnp.zeros((pad,), jnp.float32)]).reshape(NW, NCH, CHUNK)

    xp = jnp.pad(x, ((0, NPAD - N), (0, 0)))

    W2p = jnp.pad(W2, ((0, 0), (0, ZPAD - Z)))
    b2p = jnp.pad(b2, (0, ZPAD - Z)).reshape(1, ZPAD)

    degp = _make_deg_kernel()(dst3, ew3)
    mm1, mm2, fin, deg3 = _tc_calls(xp, W1, b1, W2, b2, degp)

    s1 = mm1(xp, W1, deg3)
    p = _make_agg_kernel(H)(s1, src3, dst3, ew3)
    s2 = mm2(p, s1, deg3, W2p, b1.reshape(1, H))
    q = _make_agg_kernel(ZPAD)(s2, src3, dst3, ew3)
    z = fin(q, s2, deg3, b2p)
    return z[:N, :Z]


# speedup vs baseline: 28.2426x; 1.0427x over previous
import functools

import jax
import jax.numpy as jnp
from jax import lax
from jax.experimental import pallas as pl
from jax.experimental.pallas import tpu as pltpu
from jax.experimental.pallas import tpu_sc as plsc

N = 10000
E = 320000
D_IN = 128
H = 128
Z = 64

NC = 2
NS = 16
NW = NC * NS
LANES = 16

CHUNK = 80
NCH = 128
EPW = NCH * CHUNK
EPAD = EPW * NW
K_ROWS = 4
K_EDGE = 8

ZPAD = 128
RBLK = 512
NPAD = -(-N // RBLK) * RBLK
ROWS_PER_TILE = NPAD // NS


def _worker_ids():
    cid = lax.axis_index("c")
    tid = lax.axis_index("s")
    wid = tid * NC + cid
    return cid, tid, wid


@functools.cache
def _sc_mesh():
    return plsc.VectorSubcoreMesh(core_axis_name="c", subcore_axis_name="s",
                                  num_cores=NC, num_subcores=NS)


@functools.cache
def _make_deg_kernel():
    @functools.partial(
        pl.kernel,
        out_type=jax.ShapeDtypeStruct((NC, NPAD), jnp.float32),
        scratch_types=[
            pltpu.VMEM((NCH, CHUNK), jnp.int32),
            pltpu.VMEM((NCH, CHUNK), jnp.float32),
            pltpu.VMEM((ROWS_PER_TILE,), jnp.float32),
            pltpu.VMEM_SHARED((NPAD,), jnp.float32),
        ],
        mesh=_sc_mesh(),
    )
    def _deg_kernel(dst_hbm, ew_hbm, out_hbm, idx_v, ew_v, zero_v, acc_shared):
        cid, tid, wid = _worker_ids()

        def zbody(i, _):
            zero_v[pl.ds(i * LANES, LANES)] = jnp.zeros((LANES,), jnp.float32)
            return 0

        lax.fori_loop(0, ROWS_PER_TILE // LANES, zbody, 0)
        pltpu.sync_copy(zero_v,
                        acc_shared.at[pl.ds(tid * ROWS_PER_TILE, ROWS_PER_TILE)])
        plsc.subcore_barrier()

        pltpu.sync_copy(dst_hbm.at[wid], idx_v)
        pltpu.sync_copy(ew_hbm.at[wid], ew_v)

        def body(j, _):
            pltpu.sync_copy(ew_v.at[j], acc_shared.at[idx_v.at[j]], add=True)
            return 0

        lax.fori_loop(0, NCH, body, 0)
        plsc.subcore_barrier()

        pltpu.sync_copy(
            acc_shared.at[pl.ds(tid * ROWS_PER_TILE, ROWS_PER_TILE)],
            out_hbm.at[cid, pl.ds(tid * ROWS_PER_TILE, ROWS_PER_TILE)],
        )

    return _deg_kernel


@functools.cache
def _make_agg_kernel(D):
    @functools.partial(
        pl.kernel,
        out_type=jax.ShapeDtypeStruct((NC, NPAD, D), jnp.float32),
        scratch_types=[
            pltpu.VMEM((K_EDGE, CHUNK), jnp.int32),
            pltpu.VMEM((K_EDGE, CHUNK), jnp.int32),
            pltpu.VMEM((K_EDGE, CHUNK), jnp.float32),
            pltpu.VMEM((K_ROWS, CHUNK, D), jnp.float32),
            pltpu.VMEM_SHARED((NPAD, D), jnp.float32),
            pltpu.SemaphoreType.DMA((K_EDGE,)),
            pltpu.SemaphoreType.DMA((K_ROWS,)),
            pltpu.SemaphoreType.DMA((K_ROWS,)),
        ],
        mesh=_sc_mesh(),
    )
    def agg(s_hbm, src_hbm, dst_hbm, ew_hbm, out_hbm,
            esrc, edst, eew, rows, acc_shared, e_sem, g_sem, s_sem):
        cid, tid, wid = _worker_ids()

        def eload(c, q):
            pltpu.async_copy(src_hbm.at[wid, c], esrc.at[q], e_sem.at[q])
            pltpu.async_copy(dst_hbm.at[wid, c], edst.at[q], e_sem.at[q])
            pltpu.async_copy(ew_hbm.at[wid, c], eew.at[q], e_sem.at[q])

        def ewait(q):
            pltpu.make_async_copy(src_hbm.at[wid, 0], esrc.at[q],
                                  e_sem.at[q]).wait()
            pltpu.make_async_copy(dst_hbm.at[wid, 0], edst.at[q],
                                  e_sem.at[q]).wait()
            pltpu.make_async_copy(ew_hbm.at[wid, 0], eew.at[q],
                                  e_sem.at[q]).wait()

        def gstart(q, r):
            pltpu.async_copy(s_hbm.at[esrc.at[q]], rows.at[r], g_sem.at[r])

        def gwait(q, r):
            pltpu.make_async_copy(s_hbm.at[esrc.at[q]], rows.at[r],
                                  g_sem.at[r]).wait()

        def sstart(q, r):
            pltpu.async_copy(rows.at[r], acc_shared.at[edst.at[q]],
                             s_sem.at[r], add=True)

        def swait(q, r):
            pltpu.make_async_copy(rows.at[r], acc_shared.at[edst.at[q]],
                                  s_sem.at[r]).wait()

        for q in range(5):
            eload(q, q)

        def zbody(r, _):
            for d in range(D // LANES):
                rows[0, r, pl.ds(d * LANES, LANES)] = jnp.zeros((LANES,),
                                                                jnp.float32)
            return 0

        lax.fori_loop(0, CHUNK, zbody, 0)

        def zcopy(k, _):
            pltpu.sync_copy(
                rows.at[0],
                acc_shared.at[pl.ds(tid * ROWS_PER_TILE + k * CHUNK, CHUNK)],
            )
            return 0

        lax.fori_loop(0, ROWS_PER_TILE // CHUNK, zcopy, 0)
        plsc.subcore_barrier()

        for q in range(3):
            ewait(q)
            gstart(q, q)

        lane_ids = [jnp.full((LANES, 1), r, jnp.int32) for r in range(LANES)]
        dnums = lax.GatherDimensionNumbers(
            offset_dims=(), collapsed_slice_dims=(0,), start_index_map=(0,))

        def bcast_lane(wvec, r):
            return lax.gather(wvec, lane_ids[r], dnums, (1,),
                              mode=lax.GatherScatterMode.PROMISE_IN_BOUNDS)

        def scale(q):
            def grp(g, _):
                wvec = eew[q, pl.ds(g * LANES, LANES)]
                for r in range(LANES):
                    w = bcast_lane(wvec, r)
                    row = g * LANES + r
                    for d in range(D // LANES):
                        sl = pl.ds(d * LANES, LANES)
                        rows[q % K_ROWS, row, sl] = rows[q % K_ROWS, row, sl] * w
                return 0

            lax.fori_loop(0, CHUNK // LANES, grp, 0)

        def body(k, _):
            for i in range(K_EDGE):
                c = K_EDGE * k + i
                r = i % K_ROWS
                gwait(i, r)
                scale(i)
                sstart(i, r)

                if i >= 1:
                    swait(i - 1, (i - 1) % K_ROWS)
                else:
                    @pl.when(k > 0)
                    def _sw():
                        swait((i - 1) % K_EDGE, (i - 1) % K_ROWS)

                @pl.when(c + 5 < NCH)
                def _el():
                    eload(c + 5, (i + 5) % K_EDGE)

                @pl.when(c + 3 < NCH)
                def _gs():
                    ewait((i + 3) % K_EDGE)
                    gstart((i + 3) % K_EDGE, (i + 3) % K_ROWS)

            return 0

        lax.fori_loop(0, NCH // K_EDGE, body, 0)
        swait(K_EDGE - 1, (K_EDGE - 1) % K_ROWS)
        plsc.subcore_barrier()

        pltpu.sync_copy(
            acc_shared.at[pl.ds(tid * ROWS_PER_TILE, ROWS_PER_TILE)],
            out_hbm.at[cid, pl.ds(tid * ROWS_PER_TILE, ROWS_PER_TILE)],
        )

    return agg


def _mm1_body(x_ref, w_ref, deg_ref, out_ref):
    dinv = lax.rsqrt(1.0 + deg_ref[0] + deg_ref[1])
    out_ref[...] = dinv * jnp.dot(x_ref[...], w_ref[...],
                                  preferred_element_type=jnp.float32)


def _mm2_body(p_ref, s1_ref, deg_ref, w_ref, b_ref, out_ref):
    dinv = lax.rsqrt(1.0 + deg_ref[0] + deg_ref[1])
    g = jnp.maximum(dinv * (p_ref[0] + p_ref[1] + s1_ref[...]) + b_ref[...], 0.0)
    out_ref[...] = dinv * jnp.dot(g, w_ref[...], preferred_element_type=jnp.float32)


def _fin_body(q_ref, s2_ref, deg_ref, b_ref, out_ref):
    dinv = lax.rsqrt(1.0 + deg_ref[0] + deg_ref[1])
    out_ref[...] = jnp.tanh(dinv * (q_ref[0] + q_ref[1] + s2_ref[...]) + b_ref[...])


def _tc_calls(xp, W1, b1, W2, b2, degp):
    grid = (NPAD // RBLK,)
    deg3 = degp.reshape(NC, NPAD, 1)
    dspec = pl.BlockSpec((NC, RBLK, 1), lambda i: (0, i, 0))

    mm1 = pl.pallas_call(
        _mm1_body,
        grid=grid,
        in_specs=[
            pl.BlockSpec((RBLK, D_IN), lambda i: (i, 0)),
            pl.BlockSpec((D_IN, H), lambda i: (0, 0)),
            dspec,
        ],
        out_specs=pl.BlockSpec((RBLK, H), lambda i: (i, 0)),
        out_shape=jax.ShapeDtypeStruct((NPAD, H), jnp.float32),
    )

    mm2 = pl.pallas_call(
        _mm2_body,
        grid=grid,
        in_specs=[
            pl.BlockSpec((NC, RBLK, H), lambda i: (0, i, 0)),
            pl.BlockSpec((RBLK, H), lambda i: (i, 0)),
            dspec,
            pl.BlockSpec((H, ZPAD), lambda i: (0, 0)),
            pl.BlockSpec((1, H), lambda i: (0, 0)),
        ],
        out_specs=pl.BlockSpec((RBLK, ZPAD), lambda i: (i, 0)),
        out_shape=jax.ShapeDtypeStruct((NPAD, ZPAD), jnp.float32),
    )

    fin = pl.pallas_call(
        _fin_body,
        grid=grid,
        in_specs=[
            pl.BlockSpec((NC, RBLK, ZPAD), lambda i: (0, i, 0)),
            pl.BlockSpec((RBLK, ZPAD), lambda i: (i, 0)),
            dspec,
            pl.BlockSpec((1, ZPAD), lambda i: (0, 0)),
        ],
        out_specs=pl.BlockSpec((RBLK, ZPAD), lambda i: (i, 0)),
        out_shape=jax.ShapeDtypeStruct((NPAD, ZPAD), jnp.float32),
    )
    return mm1, mm2, fin, deg3


@jax.jit
def kernel(x, edge_index, edge_weight, W1, b1, W2, b2):
    src = edge_index[0]
    dst = edge_index[1]

    pad = EPAD - E
    pad_idx = (jnp.arange(pad, dtype=jnp.int32) * 61) % N
    src3 = jnp.concatenate([src, pad_idx]).reshape(NW, NCH, CHUNK)
    dst3 = jnp.concatenate([dst, pad_idx]).reshape(NW, NCH, CHUNK)
    ew3 = jnp.concatenate(
        [edge_weight, jnp.zeros((pad,), jnp.float32)]).reshape(NW, NCH, CHUNK)

    xp = jnp.pad(x, ((0, NPAD - N), (0, 0)))

    W2p = jnp.pad(W2, ((0, 0), (0, ZPAD - Z)))
    b2p = jnp.pad(b2, (0, ZPAD - Z)).reshape(1, ZPAD)

    degp = _make_deg_kernel()(dst3, ew3)
    mm1, mm2, fin, deg3 = _tc_calls(xp, W1, b1, W2, b2, degp)

    s1 = mm1(xp, W1, deg3)
    p = _make_agg_kernel(H)(s1, src3, dst3, ew3)
    s2 = mm2(p, s1, deg3, W2p, b1.reshape(1, H))
    q = _make_agg_kernel(ZPAD)(s2, src3, dst3, ew3)
    z = fin(q, s2, deg3, b2p)
    return z[:N, :Z]


# speedup vs baseline: 28.9376x; 1.0246x over previous
import functools

import jax
import jax.numpy as jnp
from jax import lax
from jax.experimental import pallas as pl
from jax.experimental.pallas import tpu as pltpu
from jax.experimental.pallas import tpu_sc as plsc

N = 10000
E = 320000
D_IN = 128
H = 128
Z = 64

NC = 2
NS = 16
NW = NC * NS
LANES = 16

CHUNK = 80
NCH = 128
EPW = NCH * CHUNK
EPAD = EPW * NW
K_ROWS = 4
K_EDGE = 8

ZPAD = 128
RBLK = 512
NPAD = -(-N // RBLK) * RBLK
ROWS_PER_TILE = NPAD // NS


def _worker_ids():
    cid = lax.axis_index("c")
    tid = lax.axis_index("s")
    wid = tid * NC + cid
    return cid, tid, wid


@functools.cache
def _sc_mesh():
    return plsc.VectorSubcoreMesh(core_axis_name="c", subcore_axis_name="s",
                                  num_cores=NC, num_subcores=NS)


@functools.cache
def _make_deg_kernel():
    @functools.partial(
        pl.kernel,
        out_type=jax.ShapeDtypeStruct((NC, NPAD), jnp.float32),
        scratch_types=[
            pltpu.VMEM((NCH, CHUNK), jnp.int32),
            pltpu.VMEM((NCH, CHUNK), jnp.float32),
            pltpu.VMEM((ROWS_PER_TILE,), jnp.float32),
            pltpu.VMEM_SHARED((NPAD,), jnp.float32),
            pltpu.SemaphoreType.DMA,
        ],
        mesh=_sc_mesh(),
    )
    def _deg_kernel(dst_hbm, ew_hbm, out_hbm, idx_v, ew_v, zero_v, acc_shared,
                    sem):
        cid, tid, wid = _worker_ids()

        def zbody(i, _):
            zero_v[pl.ds(i * LANES, LANES)] = jnp.zeros((LANES,), jnp.float32)
            return 0

        lax.fori_loop(0, ROWS_PER_TILE // LANES, zbody, 0)
        pltpu.sync_copy(zero_v,
                        acc_shared.at[pl.ds(tid * ROWS_PER_TILE, ROWS_PER_TILE)])
        plsc.subcore_barrier()

        pltpu.sync_copy(dst_hbm.at[wid], idx_v)
        pltpu.sync_copy(ew_hbm.at[wid], ew_v)

        def body(j, _):
            pltpu.async_copy(ew_v.at[j], acc_shared.at[idx_v.at[j]], sem,
                             add=True)
            return 0

        lax.fori_loop(0, NCH, body, 0)

        def drain(j, _):
            pltpu.make_async_copy(ew_v.at[j], acc_shared.at[idx_v.at[j]],
                                  sem).wait()
            return 0

        lax.fori_loop(0, NCH, drain, 0)
        plsc.subcore_barrier()

        pltpu.sync_copy(
            acc_shared.at[pl.ds(tid * ROWS_PER_TILE, ROWS_PER_TILE)],
            out_hbm.at[cid, pl.ds(tid * ROWS_PER_TILE, ROWS_PER_TILE)],
        )

    return _deg_kernel


@functools.cache
def _make_agg_kernel(D):
    @functools.partial(
        pl.kernel,
        out_type=jax.ShapeDtypeStruct((NC, NPAD, D), jnp.float32),
        scratch_types=[
            pltpu.VMEM((K_EDGE, CHUNK), jnp.int32),
            pltpu.VMEM((K_EDGE, CHUNK), jnp.int32),
            pltpu.VMEM((K_EDGE, CHUNK), jnp.float32),
            pltpu.VMEM((K_ROWS, CHUNK, D), jnp.float32),
            pltpu.VMEM_SHARED((NPAD, D), jnp.float32),
            pltpu.SemaphoreType.DMA((K_EDGE,)),
            pltpu.SemaphoreType.DMA((K_ROWS,)),
            pltpu.SemaphoreType.DMA((K_ROWS,)),
        ],
        mesh=_sc_mesh(),
    )
    def agg(s_hbm, src_hbm, dst_hbm, ew_hbm, out_hbm,
            esrc, edst, eew, rows, acc_shared, e_sem, g_sem, s_sem):
        cid, tid, wid = _worker_ids()

        def eload(c, q):
            pltpu.async_copy(src_hbm.at[wid, c], esrc.at[q], e_sem.at[q])
            pltpu.async_copy(dst_hbm.at[wid, c], edst.at[q], e_sem.at[q])
            pltpu.async_copy(ew_hbm.at[wid, c], eew.at[q], e_sem.at[q])

        def ewait(q):
            pltpu.make_async_copy(src_hbm.at[wid, 0], esrc.at[q],
                                  e_sem.at[q]).wait()
            pltpu.make_async_copy(dst_hbm.at[wid, 0], edst.at[q],
                                  e_sem.at[q]).wait()
            pltpu.make_async_copy(ew_hbm.at[wid, 0], eew.at[q],
                                  e_sem.at[q]).wait()

        def gstart(q, r):
            pltpu.async_copy(s_hbm.at[esrc.at[q]], rows.at[r], g_sem.at[r])

        def gwait(q, r):
            pltpu.make_async_copy(s_hbm.at[esrc.at[q]], rows.at[r],
                                  g_sem.at[r]).wait()

        def sstart(q, r):
            pltpu.async_copy(rows.at[r], acc_shared.at[edst.at[q]],
                             s_sem.at[r], add=True)

        def swait(q, r):
            pltpu.make_async_copy(rows.at[r], acc_shared.at[edst.at[q]],
                                  s_sem.at[r]).wait()

        for q in range(5):
            eload(q, q)

        def zbody(r, _):
            for d in range(D // LANES):
                rows[0, r, pl.ds(d * LANES, LANES)] = jnp.zeros((LANES,),
                                                                jnp.float32)
            return 0

        lax.fori_loop(0, CHUNK, zbody, 0)

        def zcopy(k, _):
            pltpu.sync_copy(
                rows.at[0],
                acc_shared.at[pl.ds(tid * ROWS_PER_TILE + k * CHUNK, CHUNK)],
            )
            return 0

        lax.fori_loop(0, ROWS_PER_TILE // CHUNK, zcopy, 0)
        plsc.subcore_barrier()

        for q in range(3):
            ewait(q)
            gstart(q, q)

        lane_ids = [jnp.full((LANES, 1), r, jnp.int32) for r in range(LANES)]
        dnums = lax.GatherDimensionNumbers(
            offset_dims=(), collapsed_slice_dims=(0,), start_index_map=(0,))

        def bcast_lane(wvec, r):
            return lax.gather(wvec, lane_ids[r], dnums, (1,),
                              mode=lax.GatherScatterMode.PROMISE_IN_BOUNDS)

        def scale(q):
            def grp(g, _):
                wvec = eew[q, pl.ds(g * LANES, LANES)]
                for r in range(LANES):
                    w = bcast_lane(wvec, r)
                    row = g * LANES + r
                    for d in range(D // LANES):
                        sl = pl.ds(d * LANES, LANES)
                        rows[q % K_ROWS, row, sl] = rows[q % K_ROWS, row, sl] * w
                return 0

            lax.fori_loop(0, CHUNK // LANES, grp, 0)

        def body(k, _):
            for i in range(K_EDGE):
                c = K_EDGE * k + i
                r = i % K_ROWS
                gwait(i, r)
                scale(i)
                sstart(i, r)

                if i >= 1:
                    swait(i - 1, (i - 1) % K_ROWS)
                else:
                    @pl.when(k > 0)
                    def _sw():
                        swait((i - 1) % K_EDGE, (i - 1) % K_ROWS)

                @pl.when(c + 5 < NCH)
                def _el():
                    eload(c + 5, (i + 5) % K_EDGE)

                @pl.when(c + 3 < NCH)
                def _gs():
                    ewait((i + 3) % K_EDGE)
                    gstart((i + 3) % K_EDGE, (i + 3) % K_ROWS)

            return 0

        lax.fori_loop(0, NCH // K_EDGE, body, 0)
        swait(K_EDGE - 1, (K_EDGE - 1) % K_ROWS)
        plsc.subcore_barrier()

        pltpu.sync_copy(
            acc_shared.at[pl.ds(tid * ROWS_PER_TILE, ROWS_PER_TILE)],
            out_hbm.at[cid, pl.ds(tid * ROWS_PER_TILE, ROWS_PER_TILE)],
        )

    return agg


def _mm1_body(x_ref, w_ref, deg_ref, out_ref):
    dinv = lax.rsqrt(1.0 + deg_ref[0] + deg_ref[1])
    out_ref[...] = dinv * jnp.dot(x_ref[...], w_ref[...],
                                  preferred_element_type=jnp.float32)


def _mm2_body(p_ref, s1_ref, deg_ref, w_ref, b_ref, out_ref):
    dinv = lax.rsqrt(1.0 + deg_ref[0] + deg_ref[1])
    g = jnp.maximum(dinv * (p_ref[0] + p_ref[1] + s1_ref[...]) + b_ref[...], 0.0)
    out_ref[...] = dinv * jnp.dot(g, w_ref[...], preferred_element_type=jnp.float32)


def _fin_body(q_ref, s2_ref, deg_ref, b_ref, out_ref):
    dinv = lax.rsqrt(1.0 + deg_ref[0] + deg_ref[1])
    out_ref[...] = jnp.tanh(dinv * (q_ref[0] + q_ref[1] + s2_ref[...]) + b_ref[...])


def _tc_calls(xp, W1, b1, W2, b2, degp):
    grid = (NPAD // RBLK,)
    deg3 = degp.reshape(NC, NPAD, 1)
    dspec = pl.BlockSpec((NC, RBLK, 1), lambda i: (0, i, 0))

    mm1 = pl.pallas_call(
        _mm1_body,
        grid=grid,
        in_specs=[
            pl.BlockSpec((RBLK, D_IN), lambda i: (i, 0)),
            pl.BlockSpec((D_IN, H), lambda i: (0, 0)),
            dspec,
        ],
        out_specs=pl.BlockSpec((RBLK, H), lambda i: (i, 0)),
        out_shape=jax.ShapeDtypeStruct((NPAD, H), jnp.float32),
    )

    mm2 = pl.pallas_call(
        _mm2_body,
        grid=grid,
        in_specs=[
            pl.BlockSpec((NC, RBLK, H), lambda i: (0, i, 0)),
            pl.BlockSpec((RBLK, H), lambda i: (i, 0)),
            dspec,
            pl.BlockSpec((H, ZPAD), lambda i: (0, 0)),
            pl.BlockSpec((1, H), lambda i: (0, 0)),
        ],
        out_specs=pl.BlockSpec((RBLK, ZPAD), lambda i: (i, 0)),
        out_shape=jax.ShapeDtypeStruct((NPAD, ZPAD), jnp.float32),
    )

    fin = pl.pallas_call(
        _fin_body,
        grid=grid,
        in_specs=[
            pl.BlockSpec((NC, RBLK, ZPAD), lambda i: (0, i, 0)),
            pl.BlockSpec((RBLK, ZPAD), lambda i: (i, 0)),
            dspec,
            pl.BlockSpec((1, ZPAD), lambda i: (0, 0)),
        ],
        out_specs=pl.BlockSpec((RBLK, ZPAD), lambda i: (i, 0)),
        out_shape=jax.ShapeDtypeStruct((NPAD, ZPAD), jnp.float32),
    )
    return mm1, mm2, fin, deg3


@jax.jit
def kernel(x, edge_index, edge_weight, W1, b1, W2, b2):
    src = edge_index[0]
    dst = edge_index[1]

    pad = EPAD - E
    pad_idx = (jnp.arange(pad, dtype=jnp.int32) * 61) % N
    src3 = jnp.concatenate([src, pad_idx]).reshape(NW, NCH, CHUNK)
    dst3 = jnp.concatenate([dst, pad_idx]).reshape(NW, NCH, CHUNK)
    ew3 = jnp.concatenate(
        [edge_weight, jnp.zeros((pad,), jnp.float32)]).reshape(NW, NCH, CHUNK)

    xp = jnp.pad(x, ((0, NPAD - N), (0, 0)))

    W2p = jnp.pad(W2, ((0, 0), (0, ZPAD - Z)))
    b2p = jnp.pad(b2, (0, ZPAD - Z)).reshape(1, ZPAD)

    degp = _make_deg_kernel()(dst3, ew3)
    mm1, mm2, fin, deg3 = _tc_calls(xp, W1, b1, W2, b2, degp)

    s1 = mm1(xp, W1, deg3)
    p = _make_agg_kernel(H)(s1, src3, dst3, ew3)
    s2 = mm2(p, s1, deg3, W2p, b1.reshape(1, H))
    q = _make_agg_kernel(ZPAD)(s2, src3, dst3, ew3)
    z = fin(q, s2, deg3, b2p)
    return z[:N, :Z]


# speedup vs baseline: 30.3156x; 1.0476x over previous
import functools

import jax
import jax.numpy as jnp
from jax import lax
from jax.experimental import pallas as pl
from jax.experimental.pallas import tpu as pltpu
from jax.experimental.pallas import tpu_sc as plsc

N = 10000
E = 320000
D_IN = 128
H = 128
Z = 64

NC = 2
NS = 16
NW = NC * NS
LANES = 16

CHUNK = 80
NCH = 128
EPW = NCH * CHUNK
EPAD = EPW * NW
K_ROWS = 4
K_EDGE = 8

ZPAD = 128
RBLK = 1024
NPAD = -(-N // RBLK) * RBLK
ROWS_PER_TILE = NPAD // NS


def _worker_ids():
    cid = lax.axis_index("c")
    tid = lax.axis_index("s")
    wid = tid * NC + cid
    return cid, tid, wid


@functools.cache
def _sc_mesh():
    return plsc.VectorSubcoreMesh(core_axis_name="c", subcore_axis_name="s",
                                  num_cores=NC, num_subcores=NS)


@functools.cache
def _make_deg_kernel():
    @functools.partial(
        pl.kernel,
        out_type=jax.ShapeDtypeStruct((NC, NPAD), jnp.float32),
        scratch_types=[
            pltpu.VMEM((NCH, CHUNK), jnp.int32),
            pltpu.VMEM((NCH, CHUNK), jnp.float32),
            pltpu.VMEM((ROWS_PER_TILE,), jnp.float32),
            pltpu.VMEM_SHARED((NPAD,), jnp.float32),
            pltpu.SemaphoreType.DMA,
        ],
        mesh=_sc_mesh(),
    )
    def _deg_kernel(dst_hbm, ew_hbm, out_hbm, idx_v, ew_v, zero_v, acc_shared,
                    sem):
        cid, tid, wid = _worker_ids()

        def zbody(i, _):
            zero_v[pl.ds(i * LANES, LANES)] = jnp.zeros((LANES,), jnp.float32)
            return 0

        lax.fori_loop(0, ROWS_PER_TILE // LANES, zbody, 0)
        pltpu.sync_copy(zero_v,
                        acc_shared.at[pl.ds(tid * ROWS_PER_TILE, ROWS_PER_TILE)])
        plsc.subcore_barrier()

        pltpu.sync_copy(dst_hbm.at[wid], idx_v)
        pltpu.sync_copy(ew_hbm.at[wid], ew_v)

        def body(j, _):
            pltpu.async_copy(ew_v.at[j], acc_shared.at[idx_v.at[j]], sem,
                             add=True)
            return 0

        lax.fori_loop(0, NCH, body, 0)

        def drain(j, _):
            pltpu.make_async_copy(ew_v.at[j], acc_shared.at[idx_v.at[j]],
                                  sem).wait()
            return 0

        lax.fori_loop(0, NCH, drain, 0)
        plsc.subcore_barrier()

        pltpu.sync_copy(
            acc_shared.at[pl.ds(tid * ROWS_PER_TILE, ROWS_PER_TILE)],
            out_hbm.at[cid, pl.ds(tid * ROWS_PER_TILE, ROWS_PER_TILE)],
        )

    return _deg_kernel


@functools.cache
def _make_agg_kernel(D):
    @functools.partial(
        pl.kernel,
        out_type=jax.ShapeDtypeStruct((NC, NPAD, D), jnp.float32),
        scratch_types=[
            pltpu.VMEM((K_EDGE, CHUNK), jnp.int32),
            pltpu.VMEM((K_EDGE, CHUNK), jnp.int32),
            pltpu.VMEM((K_EDGE, CHUNK), jnp.float32),
            pltpu.VMEM((K_ROWS, CHUNK, D), jnp.float32),
            pltpu.VMEM_SHARED((NPAD, D), jnp.float32),
            pltpu.SemaphoreType.DMA((K_EDGE,)),
            pltpu.SemaphoreType.DMA((K_ROWS,)),
            pltpu.SemaphoreType.DMA((K_ROWS,)),
        ],
        mesh=_sc_mesh(),
    )
    def agg(s_hbm, src_hbm, dst_hbm, ew_hbm, out_hbm,
            esrc, edst, eew, rows, acc_shared, e_sem, g_sem, s_sem):
        cid, tid, wid = _worker_ids()

        def eload(c, q):
            pltpu.async_copy(src_hbm.at[wid, c], esrc.at[q], e_sem.at[q])
            pltpu.async_copy(dst_hbm.at[wid, c], edst.at[q], e_sem.at[q])
            pltpu.async_copy(ew_hbm.at[wid, c], eew.at[q], e_sem.at[q])

        def ewait(q):
            pltpu.make_async_copy(src_hbm.at[wid, 0], esrc.at[q],
                                  e_sem.at[q]).wait()
            pltpu.make_async_copy(dst_hbm.at[wid, 0], edst.at[q],
                                  e_sem.at[q]).wait()
            pltpu.make_async_copy(ew_hbm.at[wid, 0], eew.at[q],
                                  e_sem.at[q]).wait()

        def gstart(q, r):
            pltpu.async_copy(s_hbm.at[esrc.at[q]], rows.at[r], g_sem.at[r])

        def gwait(q, r):
            pltpu.make_async_copy(s_hbm.at[esrc.at[q]], rows.at[r],
                                  g_sem.at[r]).wait()

        def sstart(q, r):
            pltpu.async_copy(rows.at[r], acc_shared.at[edst.at[q]],
                             s_sem.at[r], add=True)

        def swait(q, r):
            pltpu.make_async_copy(rows.at[r], acc_shared.at[edst.at[q]],
                                  s_sem.at[r]).wait()

        for q in range(5):
            eload(q, q)

        def zbody(r, _):
            for d in range(D // LANES):
                rows[0, r, pl.ds(d * LANES, LANES)] = jnp.zeros((LANES,),
                                                                jnp.float32)
            return 0

        lax.fori_loop(0, CHUNK, zbody, 0)

        def zcopy(k, _):
            pltpu.sync_copy(
                rows.at[0],
                acc_shared.at[pl.ds(tid * ROWS_PER_TILE + k * CHUNK, CHUNK)],
            )
            return 0

        lax.fori_loop(0, ROWS_PER_TILE // CHUNK, zcopy, 0)
        plsc.subcore_barrier()

        for q in range(3):
            ewait(q)
            gstart(q, q)

        lane_ids = [jnp.full((LANES, 1), r, jnp.int32) for r in range(LANES)]
        dnums = lax.GatherDimensionNumbers(
            offset_dims=(), collapsed_slice_dims=(0,), start_index_map=(0,))

        def bcast_lane(wvec, r):
            return lax.gather(wvec, lane_ids[r], dnums, (1,),
                              mode=lax.GatherScatterMode.PROMISE_IN_BOUNDS)

        def scale(q):
            def grp(g, _):
                wvec = eew[q, pl.ds(g * LANES, LANES)]
                for r in range(LANES):
                    w = bcast_lane(wvec, r)
                    row = g * LANES + r
                    for d in range(D // LANES):
                        sl = pl.ds(d * LANES, LANES)
                        rows[q % K_ROWS, row, sl] = rows[q % K_ROWS, row, sl] * w
                return 0

            lax.fori_loop(0, CHUNK // LANES, grp, 0)

        def body(k, _):
            for i in range(K_EDGE):
                c = K_EDGE * k + i
                r = i % K_ROWS
                gwait(i, r)
                scale(i)
                sstart(i, r)

                if i >= 1:
                    swait(i - 1, (i - 1) % K_ROWS)
                else:
                    @pl.when(k > 0)
                    def _sw():
                        swait((i - 1) % K_EDGE, (i - 1) % K_ROWS)

                @pl.when(c + 5 < NCH)
                def _el():
                    eload(c + 5, (i + 5) % K_EDGE)

                @pl.when(c + 3 < NCH)
                def _gs():
                    ewait((i + 3) % K_EDGE)
                    gstart((i + 3) % K_EDGE, (i + 3) % K_ROWS)

            return 0

        lax.fori_loop(0, NCH // K_EDGE, body, 0)
        swait(K_EDGE - 1, (K_EDGE - 1) % K_ROWS)
        plsc.subcore_barrier()

        pltpu.sync_copy(
            acc_shared.at[pl.ds(tid * ROWS_PER_TILE, ROWS_PER_TILE)],
            out_hbm.at[cid, pl.ds(tid * ROWS_PER_TILE, ROWS_PER_TILE)],
        )

    return agg


def _mm1_body(x_ref, w_ref, deg_ref, out_ref):
    dinv = lax.rsqrt(1.0 + deg_ref[0] + deg_ref[1])
    out_ref[...] = dinv * jnp.dot(x_ref[...], w_ref[...],
                                  preferred_element_type=jnp.float32)


def _mm2_body(p_ref, s1_ref, deg_ref, w_ref, b_ref, out_ref):
    dinv = lax.rsqrt(1.0 + deg_ref[0] + deg_ref[1])
    g = jnp.maximum(dinv * (p_ref[0] + p_ref[1] + s1_ref[...]) + b_ref[...], 0.0)
    out_ref[...] = dinv * jnp.dot(g, w_ref[...], preferred_element_type=jnp.float32)


def _fin_body(q_ref, s2_ref, deg_ref, b_ref, out_ref):
    dinv = lax.rsqrt(1.0 + deg_ref[0] + deg_ref[1])
    out_ref[...] = jnp.tanh(dinv * (q_ref[0] + q_ref[1] + s2_ref[...]) + b_ref[...])


def _tc_calls(xp, W1, b1, W2, b2, degp):
    grid = (NPAD // RBLK,)
    deg3 = degp.reshape(NC, NPAD, 1)
    dspec = pl.BlockSpec((NC, RBLK, 1), lambda i: (0, i, 0))

    mm1 = pl.pallas_call(
        _mm1_body,
        grid=grid,
        in_specs=[
            pl.BlockSpec((RBLK, D_IN), lambda i: (i, 0)),
            pl.BlockSpec((D_IN, H), lambda i: (0, 0)),
            dspec,
        ],
        out_specs=pl.BlockSpec((RBLK, H), lambda i: (i, 0)),
        out_shape=jax.ShapeDtypeStruct((NPAD, H), jnp.float32),
    )

    mm2 = pl.pallas_call(
        _mm2_body,
        grid=grid,
        in_specs=[
            pl.BlockSpec((NC, RBLK, H), lambda i: (0, i, 0)),
            pl.BlockSpec((RBLK, H), lambda i: (i, 0)),
            dspec,
            pl.BlockSpec((H, ZPAD), lambda i: (0, 0)),
            pl.BlockSpec((1, H), lambda i: (0, 0)),
        ],
        out_specs=pl.BlockSpec((RBLK, ZPAD), lambda i: (i, 0)),
        out_shape=jax.ShapeDtypeStruct((NPAD, ZPAD), jnp.float32),
    )

    fin = pl.pallas_call(
        _fin_body,
        grid=grid,
        in_specs=[
            pl.BlockSpec((NC, RBLK, ZPAD), lambda i: (0, i, 0)),
            pl.BlockSpec((RBLK, ZPAD), lambda i: (i, 0)),
            dspec,
            pl.BlockSpec((1, ZPAD), lambda i: (0, 0)),
        ],
        out_specs=pl.BlockSpec((RBLK, ZPAD), lambda i: (i, 0)),
        out_shape=jax.ShapeDtypeStruct((NPAD, ZPAD), jnp.float32),
    )
    return mm1, mm2, fin, deg3


@jax.jit
def kernel(x, edge_index, edge_weight, W1, b1, W2, b2):
    src = edge_index[0]
    dst = edge_index[1]

    pad = EPAD - E
    pad_idx = (jnp.arange(pad, dtype=jnp.int32) * 61) % N
    src3 = jnp.concatenate([src, pad_idx]).reshape(NW, NCH, CHUNK)
    dst3 = jnp.concatenate([dst, pad_idx]).reshape(NW, NCH, CHUNK)
    ew3 = jnp.concatenate(
        [edge_weight, jnp.zeros((pad,), jnp.float32)]).reshape(NW, NCH, CHUNK)

    xp = jnp.pad(x, ((0, NPAD - N), (0, 0)))

    W2p = jnp.pad(W2, ((0, 0), (0, ZPAD - Z)))
    b2p = jnp.pad(b2, (0, ZPAD - Z)).reshape(1, ZPAD)

    degp = _make_deg_kernel()(dst3, ew3)
    mm1, mm2, fin, deg3 = _tc_calls(xp, W1, b1, W2, b2, degp)

    s1 = mm1(xp, W1, deg3)
    p = _make_agg_kernel(H)(s1, src3, dst3, ew3)
    s2 = mm2(p, s1, deg3, W2p, b1.reshape(1, H))
    q = _make_agg_kernel(ZPAD)(s2, src3, dst3, ew3)
    z = fin(q, s2, deg3, b2p)
    return z[:N, :Z]


# speedup vs baseline: 30.9394x; 1.0206x over previous
import functools

import jax
import jax.numpy as jnp
from jax import lax
from jax.experimental import pallas as pl
from jax.experimental.pallas import tpu as pltpu
from jax.experimental.pallas import tpu_sc as plsc

N = 10000
E = 320000
D_IN = 128
H = 128
Z = 64

NC = 2
NS = 16
NW = NC * NS
LANES = 16

CHUNK = 80
NCH = 128
EPW = NCH * CHUNK
EPAD = EPW * NW
K_ROWS = 4
K_EDGE = 8

ZPAD = 128
RBLK = 2048
NPAD = -(-N // RBLK) * RBLK
ROWS_PER_TILE = NPAD // NS


def _worker_ids():
    cid = lax.axis_index("c")
    tid = lax.axis_index("s")
    wid = tid * NC + cid
    return cid, tid, wid


@functools.cache
def _sc_mesh():
    return plsc.VectorSubcoreMesh(core_axis_name="c", subcore_axis_name="s",
                                  num_cores=NC, num_subcores=NS)


@functools.cache
def _make_deg_kernel():
    @functools.partial(
        pl.kernel,
        out_type=jax.ShapeDtypeStruct((NC, NPAD), jnp.float32),
        scratch_types=[
            pltpu.VMEM((NCH, CHUNK), jnp.int32),
            pltpu.VMEM((NCH, CHUNK), jnp.float32),
            pltpu.VMEM((ROWS_PER_TILE,), jnp.float32),
            pltpu.VMEM_SHARED((NPAD,), jnp.float32),
            pltpu.SemaphoreType.DMA,
        ],
        mesh=_sc_mesh(),
    )
    def _deg_kernel(dst_hbm, ew_hbm, out_hbm, idx_v, ew_v, zero_v, acc_shared,
                    sem):
        cid, tid, wid = _worker_ids()

        def zbody(i, _):
            zero_v[pl.ds(i * LANES, LANES)] = jnp.zeros((LANES,), jnp.float32)
            return 0

        lax.fori_loop(0, ROWS_PER_TILE // LANES, zbody, 0)
        pltpu.sync_copy(zero_v,
                        acc_shared.at[pl.ds(tid * ROWS_PER_TILE, ROWS_PER_TILE)])
        plsc.subcore_barrier()

        pltpu.sync_copy(dst_hbm.at[wid], idx_v)
        pltpu.sync_copy(ew_hbm.at[wid], ew_v)

        def body(j, _):
            pltpu.async_copy(ew_v.at[j], acc_shared.at[idx_v.at[j]], sem,
                             add=True)
            return 0

        lax.fori_loop(0, NCH, body, 0)

        def drain(j, _):
            pltpu.make_async_copy(ew_v.at[j], acc_shared.at[idx_v.at[j]],
                                  sem).wait()
            return 0

        lax.fori_loop(0, NCH, drain, 0)
        plsc.subcore_barrier()

        pltpu.sync_copy(
            acc_shared.at[pl.ds(tid * ROWS_PER_TILE, ROWS_PER_TILE)],
            out_hbm.at[cid, pl.ds(tid * ROWS_PER_TILE, ROWS_PER_TILE)],
        )

    return _deg_kernel


@functools.cache
def _make_agg_kernel(D):
    @functools.partial(
        pl.kernel,
        out_type=jax.ShapeDtypeStruct((NC, NPAD, D), jnp.float32),
        scratch_types=[
            pltpu.VMEM((K_EDGE, CHUNK), jnp.int32),
            pltpu.VMEM((K_EDGE, CHUNK), jnp.int32),
            pltpu.VMEM((K_EDGE, CHUNK), jnp.float32),
            pltpu.VMEM((K_ROWS, CHUNK, D), jnp.float32),
            pltpu.VMEM_SHARED((NPAD, D), jnp.float32),
            pltpu.SemaphoreType.DMA((K_EDGE,)),
            pltpu.SemaphoreType.DMA((K_ROWS,)),
            pltpu.SemaphoreType.DMA((K_ROWS,)),
        ],
        mesh=_sc_mesh(),
    )
    def agg(s_hbm, src_hbm, dst_hbm, ew_hbm, out_hbm,
            esrc, edst, eew, rows, acc_shared, e_sem, g_sem, s_sem):
        cid, tid, wid = _worker_ids()

        def eload(c, q):
            pltpu.async_copy(src_hbm.at[wid, c], esrc.at[q], e_sem.at[q])
            pltpu.async_copy(dst_hbm.at[wid, c], edst.at[q], e_sem.at[q])
            pltpu.async_copy(ew_hbm.at[wid, c], eew.at[q], e_sem.at[q])

        def ewait(q):
            pltpu.make_async_copy(src_hbm.at[wid, 0], esrc.at[q],
                                  e_sem.at[q]).wait()
            pltpu.make_async_copy(dst_hbm.at[wid, 0], edst.at[q],
                                  e_sem.at[q]).wait()
            pltpu.make_async_copy(ew_hbm.at[wid, 0], eew.at[q],
                                  e_sem.at[q]).wait()

        def gstart(q, r):
            pltpu.async_copy(s_hbm.at[esrc.at[q]], rows.at[r], g_sem.at[r])

        def gwait(q, r):
            pltpu.make_async_copy(s_hbm.at[esrc.at[q]], rows.at[r],
                                  g_sem.at[r]).wait()

        def sstart(q, r):
            pltpu.async_copy(rows.at[r], acc_shared.at[edst.at[q]],
                             s_sem.at[r], add=True)

        def swait(q, r):
            pltpu.make_async_copy(rows.at[r], acc_shared.at[edst.at[q]],
                                  s_sem.at[r]).wait()

        for q in range(5):
            eload(q, q)

        def zbody(r, _):
            for d in range(D // LANES):
                rows[0, r, pl.ds(d * LANES, LANES)] = jnp.zeros((LANES,),
                                                                jnp.float32)
            return 0

        lax.fori_loop(0, CHUNK, zbody, 0)

        def zcopy(k, _):
            pltpu.sync_copy(
                rows.at[0],
                acc_shared.at[pl.ds(tid * ROWS_PER_TILE + k * CHUNK, CHUNK)],
            )
            return 0

        lax.fori_loop(0, ROWS_PER_TILE // CHUNK, zcopy, 0)
        plsc.subcore_barrier()

        for q in range(3):
            ewait(q)
            gstart(q, q)

        lane_ids = [jnp.full((LANES, 1), r, jnp.int32) for r in range(LANES)]
        dnums = lax.GatherDimensionNumbers(
            offset_dims=(), collapsed_slice_dims=(0,), start_index_map=(0,))

        def bcast_lane(wvec, r):
            return lax.gather(wvec, lane_ids[r], dnums, (1,),
                              mode=lax.GatherScatterMode.PROMISE_IN_BOUNDS)

        def scale(q):
            def grp(g, _):
                wvec = eew[q, pl.ds(g * LANES, LANES)]
                for r in range(LANES):
                    w = bcast_lane(wvec, r)
                    row = g * LANES + r
                    for d in range(D // LANES):
                        sl = pl.ds(d * LANES, LANES)
                        rows[q % K_ROWS, row, sl] = rows[q % K_ROWS, row, sl] * w
                return 0

            lax.fori_loop(0, CHUNK // LANES, grp, 0)

        def body(k, _):
            for i in range(K_EDGE):
                c = K_EDGE * k + i
                r = i % K_ROWS
                gwait(i, r)
                scale(i)
                sstart(i, r)

                if i >= 1:
                    swait(i - 1, (i - 1) % K_ROWS)
                else:
                    @pl.when(k > 0)
                    def _sw():
                        swait((i - 1) % K_EDGE, (i - 1) % K_ROWS)

                @pl.when(c + 5 < NCH)
                def _el():
                    eload(c + 5, (i + 5) % K_EDGE)

                @pl.when(c + 3 < NCH)
                def _gs():
                    ewait((i + 3) % K_EDGE)
                    gstart((i + 3) % K_EDGE, (i + 3) % K_ROWS)

            return 0

        lax.fori_loop(0, NCH // K_EDGE, body, 0)
        swait(K_EDGE - 1, (K_EDGE - 1) % K_ROWS)
        plsc.subcore_barrier()

        pltpu.sync_copy(
            acc_shared.at[pl.ds(tid * ROWS_PER_TILE, ROWS_PER_TILE)],
            out_hbm.at[cid, pl.ds(tid * ROWS_PER_TILE, ROWS_PER_TILE)],
        )

    return agg


def _mm1_body(x_ref, w_ref, deg_ref, out_ref):
    dinv = lax.rsqrt(1.0 + deg_ref[0] + deg_ref[1])
    out_ref[...] = dinv * jnp.dot(x_ref[...], w_ref[...],
                                  preferred_element_type=jnp.float32)


def _mm2_body(p_ref, s1_ref, deg_ref, w_ref, b_ref, out_ref):
    dinv = lax.rsqrt(1.0 + deg_ref[0] + deg_ref[1])
    g = jnp.maximum(dinv * (p_ref[0] + p_ref[1] + s1_ref[...]) + b_ref[...], 0.0)
    out_ref[...] = dinv * jnp.dot(g, w_ref[...], preferred_element_type=jnp.float32)


def _fin_body(q_ref, s2_ref, deg_ref, b_ref, out_ref):
    dinv = lax.rsqrt(1.0 + deg_ref[0] + deg_ref[1])
    out_ref[...] = jnp.tanh(dinv * (q_ref[0] + q_ref[1] + s2_ref[...]) + b_ref[...])


def _tc_calls(xp, W1, b1, W2, b2, degp):
    grid = (NPAD // RBLK,)
    deg3 = degp.reshape(NC, NPAD, 1)
    dspec = pl.BlockSpec((NC, RBLK, 1), lambda i: (0, i, 0))

    mm1 = pl.pallas_call(
        _mm1_body,
        grid=grid,
        in_specs=[
            pl.BlockSpec((RBLK, D_IN), lambda i: (i, 0)),
            pl.BlockSpec((D_IN, H), lambda i: (0, 0)),
            dspec,
        ],
        out_specs=pl.BlockSpec((RBLK, H), lambda i: (i, 0)),
        out_shape=jax.ShapeDtypeStruct((NPAD, H), jnp.float32),
    )

    mm2 = pl.pallas_call(
        _mm2_body,
        grid=grid,
        in_specs=[
            pl.BlockSpec((NC, RBLK, H), lambda i: (0, i, 0)),
            pl.BlockSpec((RBLK, H), lambda i: (i, 0)),
            dspec,
            pl.BlockSpec((H, ZPAD), lambda i: (0, 0)),
            pl.BlockSpec((1, H), lambda i: (0, 0)),
        ],
        out_specs=pl.BlockSpec((RBLK, ZPAD), lambda i: (i, 0)),
        out_shape=jax.ShapeDtypeStruct((NPAD, ZPAD), jnp.float32),
    )

    fin = pl.pallas_call(
        _fin_body,
        grid=grid,
        in_specs=[
            pl.BlockSpec((NC, RBLK, ZPAD), lambda i: (0, i, 0)),
            pl.BlockSpec((RBLK, ZPAD), lambda i: (i, 0)),
            dspec,
            pl.BlockSpec((1, ZPAD), lambda i: (0, 0)),
        ],
        out_specs=pl.BlockSpec((RBLK, ZPAD), lambda i: (i, 0)),
        out_shape=jax.ShapeDtypeStruct((NPAD, ZPAD), jnp.float32),
    )
    return mm1, mm2, fin, deg3


@jax.jit
def kernel(x, edge_index, edge_weight, W1, b1, W2, b2):
    src = edge_index[0]
    dst = edge_index[1]

    pad = EPAD - E
    pad_idx = (jnp.arange(pad, dtype=jnp.int32) * 61) % N
    src3 = jnp.concatenate([src, pad_idx]).reshape(NW, NCH, CHUNK)
    dst3 = jnp.concatenate([dst, pad_idx]).reshape(NW, NCH, CHUNK)
    ew3 = jnp.concatenate(
        [edge_weight, jnp.zeros((pad,), jnp.float32)]).reshape(NW, NCH, CHUNK)

    xp = jnp.pad(x, ((0, NPAD - N), (0, 0)))

    W2p = jnp.pad(W2, ((0, 0), (0, ZPAD - Z)))
    b2p = jnp.pad(b2, (0, ZPAD - Z)).reshape(1, ZPAD)

    degp = _make_deg_kernel()(dst3, ew3)
    mm1, mm2, fin, deg3 = _tc_calls(xp, W1, b1, W2, b2, degp)

    s1 = mm1(xp, W1, deg3)
    p = _make_agg_kernel(H)(s1, src3, dst3, ew3)
    s2 = mm2(p, s1, deg3, W2p, b1.reshape(1, H))
    q = _make_agg_kernel(ZPAD)(s2, src3, dst3, ew3)
    z = fin(q, s2, deg3, b2p)
    return z[:N, :Z]
